# Initial kernel scaffold; baseline (speedup 1.0000x reference)
import jax
import jax.numpy as jnp
from jax.experimental import pallas as pl


def kernel(centers, scales, quats, rho):
    raise NotImplementedError("write your pallas kernel here")


# speedup vs baseline: 101.4951x; 101.4951x over previous
import jax
import jax.numpy as jnp
from jax.experimental import pallas as pl
from jax.experimental.pallas import tpu as pltpu

N = 8192
D = H = W = 128
P = 16
NPARAM = 20
GPB = 8
NSTEP = N // GPB

_ROWS, _COLS = 64, 128


def _bf(x):
    return x.astype(jnp.bfloat16).astype(jnp.float32)


def _prep_body(cz_ref, cy_ref, cx_ref, sz_ref, sy_ref, sx_ref,
               qw_ref, qx_ref, qy_ref, qz_ref, rr_ref, ri_ref,
               par_ref, izy_ref):
    cz0, cy0, cx0 = cz_ref[...], cy_ref[...], cx_ref[...]
    sz, sy, sx = sz_ref[...], sy_ref[...], sx_ref[...]
    qw, qx, qy, qz = qw_ref[...], qx_ref[...], qy_ref[...], qz_ref[...]
    rr, ri = rr_ref[...], ri_ref[...]

    qn = jnp.maximum(jnp.sqrt(qw * qw + qx * qx + qy * qy + qz * qz), 1e-6)
    qw, qx, qy, qz = qw / qn, qx / qn, qy / qn, qz / qn
    qn2 = jnp.maximum(jnp.sqrt(qw * qw + qx * qx + qy * qy + qz * qz), 1e-6)
    w, x, y, z = qw / qn2, qx / qn2, qy / qn2, qz / qn2

    ww, xx, yy, zz = w * w, x * x, y * y, z * z
    wx, wy, wz = w * x, w * y, w * z
    xy, xz, yz = x * y, x * z, y * z
    r = [[ww + xx - yy - zz, 2 * (xy - wz), 2 * (xz + wy)],
         [2 * (xy + wz), ww - xx + yy - zz, 2 * (yz - wx)],
         [2 * (xz - wy), 2 * (yz + wx), ww - xx - yy + zz]]

    spz = jnp.exp(jnp.log(jnp.maximum(sz, 1e-6)))
    spy = jnp.exp(jnp.log(jnp.maximum(sy, 1e-6)))
    spx = jnp.exp(jnp.log(jnp.maximum(sx, 1e-6)))
    iv = [1.0 / jnp.maximum(spz, 1e-4) ** 2,
          1.0 / jnp.maximum(spy, 1e-4) ** 2,
          1.0 / jnp.maximum(spx, 1e-4) ** 2]

    rb = [[_bf(r[i][j]) for j in range(3)] for i in range(3)]
    tb = [[_bf(r[k][j] * iv[j]) for j in range(3)] for k in range(3)]
    pb = [[_bf((rb[i][0] * tb[k][0] + rb[i][1] * tb[k][1]) + rb[i][2] * tb[k][2])
           for k in range(3)] for i in range(3)]

    cvz, cvy, cvx = cz0 * 127.0, cy0 * 127.0, cx0 * 127.0
    rvz, rvy, rvx = spz * 381.0, spy * 381.0, spx * 381.0

    loz = jnp.maximum(0.0, jnp.floor(cvz - rvz))
    loy = jnp.maximum(0.0, jnp.floor(cvy - rvy))
    lox = jnp.maximum(0.0, jnp.floor(cvx - rvx))
    hiz = jnp.minimum(jnp.float32(D - 1), jnp.ceil(cvz + rvz).astype(jnp.int32).astype(jnp.float32))
    hiy = jnp.minimum(jnp.float32(H - 1), jnp.ceil(cvy + rvy).astype(jnp.int32).astype(jnp.float32))
    hix = jnp.minimum(jnp.float32(W - 1), jnp.ceil(cvx + rvx).astype(jnp.int32).astype(jnp.float32))

    z0 = jnp.minimum(loz.astype(jnp.int32), D - P)
    y0 = jnp.minimum(loy.astype(jnp.int32), H - P)

    ok = (jnp.sqrt(rr * rr + ri * ri) > 1e-6).astype(jnp.float32)
    wr = rr * ok
    wi = ri * ok

    par_ref[0] = cvz
    par_ref[1] = cvy
    par_ref[2] = cvx
    for i in range(3):
        for k in range(3):
            par_ref[3 + 3 * i + k] = pb[i][k]
    par_ref[12] = loz
    par_ref[13] = hiz
    par_ref[14] = loy
    par_ref[15] = hiy
    par_ref[16] = lox
    par_ref[17] = hix
    par_ref[18] = wr
    par_ref[19] = wi
    izy_ref[0] = z0
    izy_ref[1] = y0


def _splat_body(par_ref, izy_ref, volr_ref, voli_ref):
    @pl.when(pl.program_id(0) == 0)
    def _():
        volr_ref[...] = jnp.zeros((D, H, W), jnp.float32)
        voli_ref[...] = jnp.zeros((D, H, W), jnp.float32)

    iz16 = jax.lax.broadcasted_iota(jnp.int32, (P, 1, 1), 0).astype(jnp.float32)
    iy16 = jax.lax.broadcasted_iota(jnp.int32, (1, P, 1), 1).astype(jnp.float32)
    ixw = jax.lax.broadcasted_iota(jnp.int32, (1, 1, W), 2).astype(jnp.float32)

    for j in range(GPB):
        cvz = par_ref[0, 0, j]
        cvy = par_ref[0, 1, j]
        cvx = par_ref[0, 2, j]
        pb = [[par_ref[0, 3 + 3 * i + k, j] for k in range(3)] for i in range(3)]
        loz = par_ref[0, 12, j]
        hiz = par_ref[0, 13, j]
        loy = par_ref[0, 14, j]
        hiy = par_ref[0, 15, j]
        lox = par_ref[0, 16, j]
        hix = par_ref[0, 17, j]
        wr = par_ref[0, 18, j]
        wi = par_ref[0, 19, j]
        z0 = izy_ref[0, 0, j]
        y0 = izy_ref[0, 1, j]

        zc = iz16 + z0.astype(jnp.float32)
        yc = iy16 + y0.astype(jnp.float32)
        dz = zc - cvz
        dy = yc - cvy
        dx = ixw - cvx
        dzb, dyb, dxb = _bf(dz), _bf(dy), _bf(dx)

        u = [(dzb * pb[0][jj] + dyb * pb[1][jj]) + dxb * pb[2][jj]
             for jj in range(3)]
        expo = (dz * u[0] + dy * u[1]) + dx * u[2]
        wts = jnp.exp(-0.5 * expo)

        mzy = ((zc >= loz) & (zc <= hiz) & (yc >= loy) & (yc <= hiy)).astype(jnp.float32)
        mx = ((ixw >= lox) & (ixw <= hix)).astype(jnp.float32)
        wts = (wts * mzy) * mx

        zi = pl.ds(z0, P)
        yi = pl.ds(y0, P)
        volr_ref[zi, yi, :] = volr_ref[zi, yi, :] + wts * wr
        voli_ref[zi, yi, :] = voli_ref[zi, yi, :] + wts * wi


def kernel(centers, scales, quats, rho):
    f = jnp.float32
    comps = [centers[:, 0], centers[:, 1], centers[:, 2],
             scales[:, 0], scales[:, 1], scales[:, 2],
             quats[:, 0], quats[:, 1], quats[:, 2], quats[:, 3],
             rho[:, 0], rho[:, 1]]
    comps = [c.astype(f).reshape(_ROWS, _COLS) for c in comps]

    par, izy = pl.pallas_call(
        _prep_body,
        out_shape=(
            jax.ShapeDtypeStruct((NPARAM, _ROWS, _COLS), jnp.float32),
            jax.ShapeDtypeStruct((2, _ROWS, _COLS), jnp.int32),
        ),
    )(*comps)

    par3 = jnp.transpose(par.reshape(NPARAM, NSTEP, GPB), (1, 0, 2))
    izy3 = jnp.transpose(izy.reshape(2, NSTEP, GPB), (1, 0, 2))

    volr, voli = pl.pallas_call(
        _splat_body,
        grid=(NSTEP,),
        in_specs=[
            pl.BlockSpec((1, NPARAM, GPB), lambda i: (i, 0, 0),
                         memory_space=pltpu.SMEM),
            pl.BlockSpec((1, 2, GPB), lambda i: (i, 0, 0),
                         memory_space=pltpu.SMEM),
        ],
        out_specs=(
            pl.BlockSpec((D, H, W), lambda i: (0, 0, 0)),
            pl.BlockSpec((D, H, W), lambda i: (0, 0, 0)),
        ),
        out_shape=(
            jax.ShapeDtypeStruct((D, H, W), jnp.float32),
            jax.ShapeDtypeStruct((D, H, W), jnp.float32),
        ),
        compiler_params=pltpu.CompilerParams(
            dimension_semantics=("arbitrary",),
            vmem_limit_bytes=100 * 1024 * 1024,
        ),
    )(par3, izy3)

    return jax.lax.complex(volr, voli)
